# baseline (speedup 1.0000x reference)
import functools

import jax
import jax.numpy as jnp
from jax.experimental import pallas as pl
from jax.experimental.pallas import tpu as pltpu

SMALL = -1000.0
B, T, D, L = 128, 512, 256, 16
T_BLK = 32
N_BLK = T // T_BLK
NORM_EVERY = 4


def _crf_body(x_ref, tags_ref, seqlen_ref, wt_ref, b_ref, trans_ref,
              out_ref,
              alpha_ref, scale_ref, logz_ref, real_ref, carry_ref):
    g = pl.program_id(0)
    t_base = g * T_BLK

    seq_len = seqlen_ref[...]
    trans = trans_ref[...]
    exp_t = jnp.exp(trans)

    @pl.when(g == 0)
    def _init():
        lane = jax.lax.broadcasted_iota(jnp.int32, (B, L), 1)
        alpha_ref[...] = (lane == (L - 2)).astype(jnp.float32)
        scale_ref[...] = jnp.zeros((B, 1), jnp.float32)
        logz_ref[...] = jnp.zeros((B, 1), jnp.float32)
        real_ref[...] = jnp.zeros((B, 1), jnp.float32)
        carry_ref[...] = jnp.full((B, 1), L - 2, jnp.int32)

    x2d = x_ref[...].reshape(B * T_BLK, D)
    pred2d = jnp.dot(x2d, wt_ref[...], preferred_element_type=jnp.float32)
    pred2d = pred2d + b_ref[...]
    lane2d = jax.lax.broadcasted_iota(jnp.int32, (B * T_BLK, L), 1)
    pred2d = jnp.where(lane2d >= L - 2, SMALL, pred2d)
    pred3 = pred2d.reshape(B, T_BLK, L)

    tags = jnp.transpose(tags_ref[...]).astype(jnp.int32)
    lane3 = jax.lax.broadcasted_iota(jnp.int32, (B, T_BLK, L), 2)
    oh_cur = (lane3 == tags[:, :, None]).astype(jnp.float32)
    ptags = jnp.concatenate([carry_ref[...], tags[:, :T_BLK - 1]], axis=1)
    oh_prev = (lane3 == ptags[:, :, None]).astype(jnp.float32)
    carry_ref[...] = tags[:, T_BLK - 1:]

    emit_g = jnp.sum(pred3 * oh_cur, axis=2)
    rowvals = jnp.dot(oh_prev.reshape(B * T_BLK, L), trans,
                      preferred_element_type=jnp.float32).reshape(B, T_BLK, L)
    trans_g = jnp.sum(rowvals * oh_cur, axis=2)
    to_end = jnp.sum(oh_cur * trans[:, L - 1][None, None, :], axis=2)

    t_idx = t_base + jax.lax.broadcasted_iota(jnp.int32, (B, T_BLK), 1)
    in_seq = (t_idx < seq_len).astype(jnp.float32)
    at_last = (t_idx == seq_len - 1).astype(jnp.float32)
    tile_real = jnp.sum(in_seq * (emit_g + trans_g) + at_last * to_end,
                        axis=1, keepdims=True)
    real_ref[...] = real_ref[...] + tile_real

    alpha = alpha_ref[...]
    scale = scale_ref[...]

    mx = jnp.max(pred3, axis=(1, 2), keepdims=True)
    eexp3 = jnp.exp(pred3 - mx)
    mx2 = mx[:, :, 0]

    cap_cols = []
    nrms = []
    for i in range(T_BLK):
        a1 = jnp.dot(alpha, exp_t, preferred_element_type=jnp.float32)
        cap_cols.append(a1[:, L - 1:L])
        alpha = a1 * eexp3[:, i, :]
        if i % NORM_EVERY == NORM_EVERY - 1:
            nrm = jnp.max(alpha, axis=1, keepdims=True)
            alpha = alpha * (1.0 / nrm)
            nrms.append(nrm)

    caps_raw = jnp.concatenate(cap_cols, axis=1)
    lane32 = jax.lax.broadcasted_iota(jnp.int32, (B, T_BLK), 1)
    lognrm8 = jnp.log(jnp.concatenate(nrms, axis=1))
    grpadj = jnp.zeros((B, T_BLK), jnp.float32)
    for gi in range(len(nrms) - 1):
        boundary = (gi + 1) * NORM_EVERY
        grpadj = grpadj + jnp.where(lane32 >= boundary,
                                    lognrm8[:, gi:gi + 1], 0.0)
    caps = (scale + lane32.astype(jnp.float32) * mx2 + grpadj
            + jnp.log(caps_raw))
    logz_ref[...] = logz_ref[...] + jnp.sum(
        jnp.where(t_idx == seq_len, caps, 0.0), axis=1, keepdims=True)

    alpha_ref[...] = alpha
    scale_ref[...] = (scale + T_BLK * mx2
                      + jnp.sum(lognrm8, axis=1, keepdims=True))

    @pl.when(g == N_BLK - 1)
    def _fin():
        corr = jnp.where(seq_len == 0, trans[L - 2, L - 1], 0.0)
        out_ref[...] = jnp.sum(logz_ref[...] - real_ref[...] - corr,
                               keepdims=True)


@functools.partial(jax.jit, static_argnames=())
def kernel(input, tags, seq_len, W, b, transitions):
    tags_t = tags.T.astype(jnp.float32)
    seqlen2 = seq_len.reshape(B, 1).astype(jnp.int32)
    wt = W.astype(jnp.float32).T
    b2 = b.reshape(1, L).astype(jnp.float32)

    out = pl.pallas_call(
        _crf_body,
        grid=(N_BLK,),
        in_specs=[
            pl.BlockSpec((B, T_BLK, D), lambda g: (0, g, 0)),
            pl.BlockSpec((T_BLK, B), lambda g: (g, 0)),
            pl.BlockSpec((B, 1), lambda g: (0, 0)),
            pl.BlockSpec((D, L), lambda g: (0, 0)),
            pl.BlockSpec((1, L), lambda g: (0, 0)),
            pl.BlockSpec((L, L), lambda g: (0, 0)),
        ],
        out_specs=pl.BlockSpec((1, 1), lambda g: (0, 0)),
        out_shape=jax.ShapeDtypeStruct((1, 1), jnp.float32),
        scratch_shapes=[
            pltpu.VMEM((B, L), jnp.float32),
            pltpu.VMEM((B, 1), jnp.float32),
            pltpu.VMEM((B, 1), jnp.float32),
            pltpu.VMEM((B, 1), jnp.float32),
            pltpu.VMEM((B, 1), jnp.int32),
        ],
    )(input, tags_t, seqlen2, wt, b2, transitions)
    return out[0, 0]


# speedup vs baseline: 1.1169x; 1.1169x over previous
import functools

import jax
import jax.numpy as jnp
from jax.experimental import pallas as pl
from jax.experimental.pallas import tpu as pltpu

SMALL = -1000.0
B, T, D, L = 128, 512, 256, 16
T_BLK = 32
N_BLK = T // T_BLK
NORM_EVERY = 8


def _crf_body(x_ref, tags_ref, seqlen_ref, wt_ref, b_ref, trans_ref,
              out_ref,
              alpha_ref, scale_ref, logz_ref, real_ref, carry_ref):
    g = pl.program_id(0)
    t_base = g * T_BLK

    seq_len = seqlen_ref[...]
    trans = trans_ref[...]
    exp_t = jnp.exp(trans)

    @pl.when(g == 0)
    def _init():
        lane = jax.lax.broadcasted_iota(jnp.int32, (B, L), 1)
        alpha_ref[...] = (lane == (L - 2)).astype(jnp.float32)
        scale_ref[...] = jnp.zeros((B, 1), jnp.float32)
        logz_ref[...] = jnp.zeros((B, 1), jnp.float32)
        real_ref[...] = jnp.zeros((B, 1), jnp.float32)
        carry_ref[...] = jnp.full((B, 1), L - 2, jnp.int32)

    x2d = x_ref[...].reshape(B * T_BLK, D)
    pred2d = jnp.dot(x2d, wt_ref[...], preferred_element_type=jnp.float32)
    pred2d = pred2d + b_ref[...]
    lane2d = jax.lax.broadcasted_iota(jnp.int32, (B * T_BLK, L), 1)
    pred2d = jnp.where(lane2d >= L - 2, SMALL, pred2d)
    pred3 = pred2d.reshape(B, T_BLK, L)

    tags = jnp.transpose(tags_ref[...]).astype(jnp.int32)
    lane3 = jax.lax.broadcasted_iota(jnp.int32, (B, T_BLK, L), 2)
    oh_cur = (lane3 == tags[:, :, None]).astype(jnp.float32)
    ptags = jnp.concatenate([carry_ref[...], tags[:, :T_BLK - 1]], axis=1)
    oh_prev = (lane3 == ptags[:, :, None]).astype(jnp.float32)
    carry_ref[...] = tags[:, T_BLK - 1:]

    emit_g = jnp.sum(pred3 * oh_cur, axis=2)
    rowvals = jnp.dot(oh_prev.reshape(B * T_BLK, L), trans,
                      preferred_element_type=jnp.float32).reshape(B, T_BLK, L)
    trans_g = jnp.sum(rowvals * oh_cur, axis=2)
    to_end = jnp.sum(oh_cur * trans[:, L - 1][None, None, :], axis=2)

    t_idx = t_base + jax.lax.broadcasted_iota(jnp.int32, (B, T_BLK), 1)
    in_seq = (t_idx < seq_len).astype(jnp.float32)
    at_last = (t_idx == seq_len - 1).astype(jnp.float32)
    tile_real = jnp.sum(in_seq * (emit_g + trans_g) + at_last * to_end,
                        axis=1, keepdims=True)
    real_ref[...] = real_ref[...] + tile_real

    alpha = alpha_ref[...]
    scale = scale_ref[...]

    eexp3 = jnp.exp(pred3)

    cap_cols = []
    nrms = []
    for i in range(T_BLK):
        a1 = jnp.dot(alpha, exp_t, preferred_element_type=jnp.float32)
        cap_cols.append(a1[:, L - 1:L])
        alpha = a1 * eexp3[:, i, :]
        if i % NORM_EVERY == NORM_EVERY - 1:
            nrm = jnp.max(alpha, axis=1, keepdims=True)
            alpha = alpha * (1.0 / nrm)
            nrms.append(nrm)

    caps_raw = jnp.concatenate(cap_cols, axis=1)
    lane32 = jax.lax.broadcasted_iota(jnp.int32, (B, T_BLK), 1)
    lognrm8 = jnp.log(jnp.concatenate(nrms, axis=1))
    grpadj = jnp.zeros((B, T_BLK), jnp.float32)
    for gi in range(len(nrms) - 1):
        boundary = (gi + 1) * NORM_EVERY
        grpadj = grpadj + jnp.where(lane32 >= boundary,
                                    lognrm8[:, gi:gi + 1], 0.0)
    caps = scale + grpadj + jnp.log(caps_raw)
    logz_ref[...] = logz_ref[...] + jnp.sum(
        jnp.where(t_idx == seq_len, caps, 0.0), axis=1, keepdims=True)

    alpha_ref[...] = alpha
    scale_ref[...] = scale + jnp.sum(lognrm8, axis=1, keepdims=True)

    @pl.when(g == N_BLK - 1)
    def _fin():
        corr = jnp.where(seq_len == 0, trans[L - 2, L - 1], 0.0)
        out_ref[...] = jnp.sum(logz_ref[...] - real_ref[...] - corr,
                               keepdims=True)


@functools.partial(jax.jit, static_argnames=())
def kernel(input, tags, seq_len, W, b, transitions):
    tags_t = tags.T.astype(jnp.float32)
    seqlen2 = seq_len.reshape(B, 1).astype(jnp.int32)
    wt = W.astype(jnp.float32).T
    b2 = b.reshape(1, L).astype(jnp.float32)

    out = pl.pallas_call(
        _crf_body,
        grid=(N_BLK,),
        in_specs=[
            pl.BlockSpec((B, T_BLK, D), lambda g: (0, g, 0)),
            pl.BlockSpec((T_BLK, B), lambda g: (g, 0)),
            pl.BlockSpec((B, 1), lambda g: (0, 0)),
            pl.BlockSpec((D, L), lambda g: (0, 0)),
            pl.BlockSpec((1, L), lambda g: (0, 0)),
            pl.BlockSpec((L, L), lambda g: (0, 0)),
        ],
        out_specs=pl.BlockSpec((1, 1), lambda g: (0, 0)),
        out_shape=jax.ShapeDtypeStruct((1, 1), jnp.float32),
        scratch_shapes=[
            pltpu.VMEM((B, L), jnp.float32),
            pltpu.VMEM((B, 1), jnp.float32),
            pltpu.VMEM((B, 1), jnp.float32),
            pltpu.VMEM((B, 1), jnp.float32),
            pltpu.VMEM((B, 1), jnp.int32),
        ],
    )(input, tags_t, seqlen2, wt, b2, transitions)
    return out[0, 0]


# speedup vs baseline: 1.1860x; 1.0619x over previous
import functools

import jax
import jax.numpy as jnp
from jax.experimental import pallas as pl
from jax.experimental.pallas import tpu as pltpu

SMALL = -1000.0
B, T, D, L = 128, 512, 256, 16
T_BLK = 32
N_BLK = T // T_BLK
NORM_EVERY = 16


def _crf_body(x_ref, tags_ref, seqlen_ref, wt_ref, b_ref, trans_ref,
              out_ref,
              alpha_ref, scale_ref, logz_ref, real_ref, carry_ref):
    g = pl.program_id(0)
    t_base = g * T_BLK

    seq_len = seqlen_ref[...]
    trans = trans_ref[...]
    exp_t = jnp.exp(trans)

    @pl.when(g == 0)
    def _init():
        lane = jax.lax.broadcasted_iota(jnp.int32, (B, L), 1)
        alpha_ref[...] = (lane == (L - 2)).astype(jnp.float32)
        scale_ref[...] = jnp.zeros((B, 1), jnp.float32)
        logz_ref[...] = jnp.zeros((B, 1), jnp.float32)
        real_ref[...] = jnp.zeros((B, 1), jnp.float32)
        carry_ref[...] = jnp.full((B, 1), L - 2, jnp.int32)

    x2d = x_ref[...].reshape(B * T_BLK, D)
    pred2d = jnp.dot(x2d, wt_ref[...], preferred_element_type=jnp.float32)
    pred2d = pred2d + b_ref[...]
    lane2d = jax.lax.broadcasted_iota(jnp.int32, (B * T_BLK, L), 1)
    pred2d = jnp.where(lane2d >= L - 2, SMALL, pred2d)
    pred3 = pred2d.reshape(B, T_BLK, L)

    tags = jnp.transpose(tags_ref[...]).astype(jnp.int32)
    lane3 = jax.lax.broadcasted_iota(jnp.int32, (B, T_BLK, L), 2)
    oh_cur = (lane3 == tags[:, :, None]).astype(jnp.float32)
    ptags = jnp.concatenate([carry_ref[...], tags[:, :T_BLK - 1]], axis=1)
    oh_prev = (lane3 == ptags[:, :, None]).astype(jnp.float32)
    carry_ref[...] = tags[:, T_BLK - 1:]

    emit_g = jnp.sum(pred3 * oh_cur, axis=2)
    rowvals = jnp.dot(oh_prev.reshape(B * T_BLK, L), trans,
                      preferred_element_type=jnp.float32).reshape(B, T_BLK, L)
    trans_g = jnp.sum(rowvals * oh_cur, axis=2)
    to_end = jnp.sum(oh_cur * trans[:, L - 1][None, None, :], axis=2)

    t_idx = t_base + jax.lax.broadcasted_iota(jnp.int32, (B, T_BLK), 1)
    in_seq = (t_idx < seq_len).astype(jnp.float32)
    at_last = (t_idx == seq_len - 1).astype(jnp.float32)
    tile_real = jnp.sum(in_seq * (emit_g + trans_g) + at_last * to_end,
                        axis=1, keepdims=True)
    real_ref[...] = real_ref[...] + tile_real

    alpha = alpha_ref[...]
    scale = scale_ref[...]

    eexp3 = jnp.exp(pred3)

    cap_cols = []
    nrms = []
    for i in range(T_BLK):
        a1 = jnp.dot(alpha, exp_t, preferred_element_type=jnp.float32)
        cap_cols.append(a1[:, L - 1:L])
        alpha = a1 * eexp3[:, i, :]
        if i % NORM_EVERY == NORM_EVERY - 1:
            nrm = jnp.max(alpha, axis=1, keepdims=True)
            alpha = alpha * (1.0 / nrm)
            nrms.append(nrm)

    caps_raw = jnp.concatenate(cap_cols, axis=1)
    lane32 = jax.lax.broadcasted_iota(jnp.int32, (B, T_BLK), 1)
    lognrm8 = jnp.log(jnp.concatenate(nrms, axis=1))
    grpadj = jnp.zeros((B, T_BLK), jnp.float32)
    for gi in range(len(nrms) - 1):
        boundary = (gi + 1) * NORM_EVERY
        grpadj = grpadj + jnp.where(lane32 >= boundary,
                                    lognrm8[:, gi:gi + 1], 0.0)
    caps = scale + grpadj + jnp.log(caps_raw)
    logz_ref[...] = logz_ref[...] + jnp.sum(
        jnp.where(t_idx == seq_len, caps, 0.0), axis=1, keepdims=True)

    alpha_ref[...] = alpha
    scale_ref[...] = scale + jnp.sum(lognrm8, axis=1, keepdims=True)

    @pl.when(g == N_BLK - 1)
    def _fin():
        corr = jnp.where(seq_len == 0, trans[L - 2, L - 1], 0.0)
        out_ref[...] = jnp.sum(logz_ref[...] - real_ref[...] - corr,
                               keepdims=True)


@functools.partial(jax.jit, static_argnames=())
def kernel(input, tags, seq_len, W, b, transitions):
    tags_t = tags.T.astype(jnp.float32)
    seqlen2 = seq_len.reshape(B, 1).astype(jnp.int32)
    wt = W.astype(jnp.float32).T
    b2 = b.reshape(1, L).astype(jnp.float32)

    out = pl.pallas_call(
        _crf_body,
        grid=(N_BLK,),
        in_specs=[
            pl.BlockSpec((B, T_BLK, D), lambda g: (0, g, 0)),
            pl.BlockSpec((T_BLK, B), lambda g: (g, 0)),
            pl.BlockSpec((B, 1), lambda g: (0, 0)),
            pl.BlockSpec((D, L), lambda g: (0, 0)),
            pl.BlockSpec((1, L), lambda g: (0, 0)),
            pl.BlockSpec((L, L), lambda g: (0, 0)),
        ],
        out_specs=pl.BlockSpec((1, 1), lambda g: (0, 0)),
        out_shape=jax.ShapeDtypeStruct((1, 1), jnp.float32),
        scratch_shapes=[
            pltpu.VMEM((B, L), jnp.float32),
            pltpu.VMEM((B, 1), jnp.float32),
            pltpu.VMEM((B, 1), jnp.float32),
            pltpu.VMEM((B, 1), jnp.float32),
            pltpu.VMEM((B, 1), jnp.int32),
        ],
    )(input, tags_t, seqlen2, wt, b2, transitions)
    return out[0, 0]
